# baseline (speedup 1.0000x reference)
import functools
import math

import jax
import jax.numpy as jnp
from jax import lax
from jax.experimental import pallas as pl
from jax.experimental.pallas import tpu as pltpu
from jax.experimental.pallas import tpu_sc as plsc

D_MODEL = 64
SCALE = math.sqrt(D_MODEL)
LANES = 16
VOCAB = 1000000

NUM_CORES = 2
NUM_SUBCORES = 16
NW = NUM_CORES * NUM_SUBCORES

B_TOTAL = 4096 * 200
B_PER_W = B_TOTAL // NW
CHUNK = 256
N_CHUNKS = B_PER_W // CHUNK
NGBUF = 2
NSBUF = 2
N_GROUPS = N_CHUNKS // NGBUF
W = 2 * D_MODEL


@functools.partial(
    pl.kernel,
    out_type=jax.ShapeDtypeStruct((B_TOTAL, W), jnp.float32),
    mesh=plsc.VectorSubcoreMesh(core_axis_name="c", subcore_axis_name="s"),
    compiler_params=pltpu.CompilerParams(use_tc_tiling_on_sc=False),
    scratch_types=(
        [pltpu.VMEM((N_CHUNKS, CHUNK), jnp.int32)]
        + [pltpu.VMEM((CHUNK, D_MODEL), jnp.float32) for _ in range(NGBUF)]
        + [pltpu.VMEM((CHUNK, D_MODEL), jnp.float32) for _ in range(NSBUF)]
        + [pltpu.SemaphoreType.DMA for _ in range(NGBUF + NSBUF)]
    ),
)
def _embed(table_hbm, idx_hbm, out_hbm, idx_v, *bufs_and_sems):
    gbufs = bufs_and_sems[0:NGBUF]
    sbufs = bufs_and_sems[NGBUF:NGBUF + NSBUF]
    gsems = bufs_and_sems[NGBUF + NSBUF:2 * NGBUF + NSBUF]
    ssems = bufs_and_sems[2 * NGBUF + NSBUF:2 * (NGBUF + NSBUF)]

    wid = lax.axis_index("s") * NUM_CORES + lax.axis_index("c")
    base = wid * B_PER_W

    pltpu.sync_copy(idx_hbm.at[wid], idx_v)

    for b in range(NGBUF):
        pltpu.async_copy(table_hbm.at[idx_v.at[b]], gbufs[b], gsems[b])

    def group_body(g, carry):
        for b in range(NGBUF):
            c = g * NGBUF + b
            s = b % NSBUF

            pltpu.make_async_copy(
                table_hbm.at[idx_v.at[c]], gbufs[b], gsems[b]
            ).wait()

            @pl.when((g > 0) | (b >= NSBUF))
            def _():
                pltpu.make_async_copy(
                    sbufs[s],
                    out_hbm.at[pl.ds(base + (c - NSBUF) * CHUNK, CHUNK),
                               pl.ds(0, D_MODEL)],
                    ssems[s],
                ).wait()

            def row4_body(r, c2):
                for rr in range(4):
                    i = r * 4 + rr
                    for j in range(D_MODEL // LANES):
                        sl = pl.ds(j * LANES, LANES)
                        sbufs[s][i, sl] = gbufs[b][i, sl] * SCALE
                return c2

            lax.fori_loop(0, CHUNK // 4, row4_body, 0)

            @pl.when(g < N_GROUPS - 1)
            def _():
                pltpu.async_copy(
                    table_hbm.at[idx_v.at[c + NGBUF]], gbufs[b], gsems[b]
                )

            pltpu.async_copy(
                sbufs[s],
                out_hbm.at[pl.ds(base + c * CHUNK, CHUNK), pl.ds(0, D_MODEL)],
                ssems[s],
            )
        return carry

    lax.fori_loop(0, N_GROUPS, group_body, 0)

    for s in range(NSBUF):
        c = N_CHUNKS - NSBUF + s
        pltpu.make_async_copy(
            sbufs[s],
            out_hbm.at[pl.ds(base + c * CHUNK, CHUNK), pl.ds(0, D_MODEL)],
            ssems[s],
        ).wait()


def kernel(data, embedding_weight):
    idx = data.reshape(NW, N_CHUNKS, CHUNK).astype(jnp.int32)
    out2 = _embed(embedding_weight, idx)
    return out2.reshape(data.shape[0], data.shape[1], W)[..., :D_MODEL]


# speedup vs baseline: 1.0032x; 1.0032x over previous
import functools
import math

import jax
import jax.numpy as jnp
from jax import lax
from jax.experimental import pallas as pl
from jax.experimental.pallas import tpu as pltpu
from jax.experimental.pallas import tpu_sc as plsc

D_MODEL = 64
SCALE = math.sqrt(D_MODEL)
LANES = 16
VOCAB = 1000000

NUM_CORES = 2
NUM_SUBCORES = 16
NW = NUM_CORES * NUM_SUBCORES

B_TOTAL = 4096 * 200
B_PER_W = B_TOTAL // NW
CHUNK = 128
N_CHUNKS = B_PER_W // CHUNK
NGBUF = 4
NSBUF = 4
N_GROUPS = N_CHUNKS // NGBUF
W = 2 * D_MODEL


@functools.partial(
    pl.kernel,
    out_type=jax.ShapeDtypeStruct((B_TOTAL, W), jnp.float32),
    mesh=plsc.VectorSubcoreMesh(core_axis_name="c", subcore_axis_name="s"),
    compiler_params=pltpu.CompilerParams(use_tc_tiling_on_sc=False),
    scratch_types=(
        [pltpu.VMEM((N_CHUNKS, CHUNK), jnp.int32)]
        + [pltpu.VMEM((CHUNK, D_MODEL), jnp.float32) for _ in range(NGBUF)]
        + [pltpu.VMEM((CHUNK, D_MODEL), jnp.float32) for _ in range(NSBUF)]
        + [pltpu.SemaphoreType.DMA for _ in range(NGBUF + NSBUF)]
    ),
)
def _embed(table_hbm, idx_hbm, out_hbm, idx_v, *bufs_and_sems):
    gbufs = bufs_and_sems[0:NGBUF]
    sbufs = bufs_and_sems[NGBUF:NGBUF + NSBUF]
    gsems = bufs_and_sems[NGBUF + NSBUF:2 * NGBUF + NSBUF]
    ssems = bufs_and_sems[2 * NGBUF + NSBUF:2 * (NGBUF + NSBUF)]

    wid = lax.axis_index("s") * NUM_CORES + lax.axis_index("c")
    base = wid * B_PER_W

    pltpu.sync_copy(idx_hbm.at[wid], idx_v)

    for b in range(NGBUF):
        pltpu.async_copy(table_hbm.at[idx_v.at[b]], gbufs[b], gsems[b])

    def group_body(g, carry):
        for b in range(NGBUF):
            c = g * NGBUF + b
            s = b % NSBUF

            pltpu.make_async_copy(
                table_hbm.at[idx_v.at[c]], gbufs[b], gsems[b]
            ).wait()

            @pl.when((g > 0) | (b >= NSBUF))
            def _():
                pltpu.make_async_copy(
                    sbufs[s],
                    out_hbm.at[pl.ds(base + (c - NSBUF) * CHUNK, CHUNK),
                               pl.ds(0, D_MODEL)],
                    ssems[s],
                ).wait()

            def row4_body(r, c2):
                for rr in range(4):
                    i = r * 4 + rr
                    for j in range(D_MODEL // LANES):
                        sl = pl.ds(j * LANES, LANES)
                        sbufs[s][i, sl] = gbufs[b][i, sl] * SCALE
                return c2

            lax.fori_loop(0, CHUNK // 4, row4_body, 0)

            @pl.when(g < N_GROUPS - 1)
            def _():
                pltpu.async_copy(
                    table_hbm.at[idx_v.at[c + NGBUF]], gbufs[b], gsems[b]
                )

            pltpu.async_copy(
                sbufs[s],
                out_hbm.at[pl.ds(base + c * CHUNK, CHUNK), pl.ds(0, D_MODEL)],
                ssems[s],
            )
        return carry

    lax.fori_loop(0, N_GROUPS, group_body, 0)

    for s in range(NSBUF):
        c = N_CHUNKS - NSBUF + s
        pltpu.make_async_copy(
            sbufs[s],
            out_hbm.at[pl.ds(base + c * CHUNK, CHUNK), pl.ds(0, D_MODEL)],
            ssems[s],
        ).wait()


def kernel(data, embedding_weight):
    idx = data.reshape(NW, N_CHUNKS, CHUNK).astype(jnp.int32)
    out2 = _embed(embedding_weight, idx)
    return out2.reshape(data.shape[0], data.shape[1], W)[..., :D_MODEL]
